# baseline (speedup 1.0000x reference)
import functools

import jax
import jax.numpy as jnp
from jax.experimental import pallas as pl
from jax.experimental.pallas import tpu as pltpu

_B, _NQ, _NS, _D = 16, 512, 2048, 64
_K = 128


def _maxes_kernel(q_ref, d_ref, n_ref, rm_ref, cm_ref):
    qb = q_ref[0]

    def maxes(db, side):
        s = jax.lax.dot_general(
            db, qb, (((1,), (1,)), ((), ())),
            preferred_element_type=jnp.float32)
        rm_ref[0, side:side + 1, :] = jnp.max(s, axis=0, keepdims=True)
        s2 = jax.lax.dot_general(
            qb, db, (((1,), (1,)), ((), ())),
            preferred_element_type=jnp.float32)
        cm_ref[0, side:side + 1, :] = jnp.max(s2, axis=0, keepdims=True)

    maxes(d_ref[0], 0)
    maxes(n_ref[0], 1)


def _count16(ref, cand_i32):
    cand = cand_i32.astype(jnp.int16)
    x = (ref[...] >= cand).astype(jnp.int16)
    r = x.shape[0]
    while r > 1:
        h = r // 2
        x = x[:h] + x[h:]
        r = h
    return jnp.sum(x.astype(jnp.int32))


def _bisect16_multi(refs, needs):
    ts = tuple(
        jnp.where(_count16(r, jnp.int32(0)) >= n,
                  jnp.int32(0), jnp.int32(-32768))
        for r, n in zip(refs, needs))

    def body(i, ts):
        bit = jax.lax.shift_left(jnp.int32(1), jnp.int32(14) - i)
        cands = [t | bit for t in ts]
        return tuple(
            jnp.where(_count16(r, c) >= n, c, t)
            for r, c, n, t in zip(refs, cands, needs, ts))

    return jax.lax.fori_loop(0, 15, body, ts)


def _side_keys(qc, dc, hi_ref, lo_ref):
    s = jax.lax.dot_general(
        qc, dc, (((1,), (1,)), ((), ())),
        preferred_element_type=jnp.float32)

    bits = jax.lax.bitcast_convert_type(s, jnp.int32)
    keys = jnp.where(bits < 0, bits ^ jnp.int32(0x7FFFFFFF), bits)
    hi_ref[...] = jax.lax.shift_right_arithmetic(keys, 16).astype(jnp.int16)
    lo_ref[...] = ((keys & jnp.int32(0xFFFF)) ^ jnp.int32(0x8000)).astype(
        jnp.int16)


def _qual_score(rv, tkey):
    rbits = jax.lax.bitcast_convert_type(rv, jnp.int32)
    rkeys = jnp.where(rbits < 0, rbits ^ jnp.int32(0x7FFFFFFF), rbits)
    qual = (rkeys >= tkey).astype(jnp.float32)
    z = jnp.maximum(jnp.sum(qual), 0.001)
    numer = jnp.sum(qual * jnp.maximum(rv, 0.0))
    return numer / z


def _score_kernel(qc_ref, dcd_ref, dcn_ref, rv_ref, out_ref, *scr):
    for b in range(2):
        _side_keys(qc_ref[b, 0], dcd_ref[b], scr[4 * b + 0], scr[4 * b + 1])
        _side_keys(qc_ref[b, 1], dcn_ref[b], scr[4 * b + 2], scr[4 * b + 3])
    his = [scr[0], scr[2], scr[4], scr[6]]
    los = [scr[1], scr[3], scr[5], scr[7]]

    kk = jnp.int32(_K)
    hs = _bisect16_multi(his, (kk, kk, kk, kk))
    needs = tuple(kk - _count16(h, hs[j] + 1)
                  for j, h in enumerate(his))

    for j in range(4):
        los[j][...] = jnp.where(his[j][...] == hs[j].astype(jnp.int16),
                                los[j][...], jnp.int16(-32768))
    ls = _bisect16_multi(los, needs)

    tkeys = [jax.lax.shift_left(hs[j], 16) | ((ls[j] ^ jnp.int32(0x8000))
                                              & jnp.int32(0xFFFF))
             for j in range(4)]

    sp = jnp.zeros((8, 128), jnp.float32)
    for b in range(2):
        pos = _qual_score(rv_ref[b, 0:1, :], tkeys[2 * b])
        neg = _qual_score(rv_ref[b, 1:2, :], tkeys[2 * b + 1])
        diff = neg - pos
        sp += jnp.maximum(diff, 0.0) + jnp.log1p(jnp.exp(-jnp.abs(diff)))

    @pl.when(pl.program_id(0) == 0)
    def _():
        out_ref[...] = jnp.zeros((8, 128), jnp.float32)

    out_ref[...] += sp / _B


@functools.partial(jax.jit)
def kernel(query_embeddings, doc_embeddings, neg_doc_embeddings):
    rm, cm = pl.pallas_call(
        _maxes_kernel,
        grid=(_B,),
        in_specs=[
            pl.BlockSpec((1, _NQ, _D), lambda i: (i, 0, 0)),
            pl.BlockSpec((1, _NS, _D), lambda i: (i, 0, 0)),
            pl.BlockSpec((1, _NS, _D), lambda i: (i, 0, 0)),
        ],
        out_specs=[
            pl.BlockSpec((1, 2, _NQ), lambda i: (i, 0, 0)),
            pl.BlockSpec((1, 2, _NS), lambda i: (i, 0, 0)),
        ],
        out_shape=[
            jax.ShapeDtypeStruct((_B, 2, _NQ), jnp.float32),
            jax.ShapeDtypeStruct((_B, 2, _NS), jnp.float32),
        ],
    )(query_embeddings, doc_embeddings, neg_doc_embeddings)

    rv, idx = jax.lax.top_k(rm, _K)
    _, cidx = jax.lax.top_k(cm, _K)
    qc = jnp.take_along_axis(query_embeddings[:, None], idx[..., None],
                             axis=2)
    dcd = jnp.take_along_axis(doc_embeddings, cidx[:, 0, :, None], axis=1)
    dcn = jnp.take_along_axis(neg_doc_embeddings, cidx[:, 1, :, None], axis=1)

    loss = pl.pallas_call(
        _score_kernel,
        grid=(_B // 2,),
        in_specs=[
            pl.BlockSpec((2, 2, _K, _D), lambda i: (i, 0, 0, 0)),
            pl.BlockSpec((2, _K, _D), lambda i: (i, 0, 0)),
            pl.BlockSpec((2, _K, _D), lambda i: (i, 0, 0)),
            pl.BlockSpec((2, 2, _K), lambda i: (i, 0, 0)),
        ],
        out_specs=pl.BlockSpec((8, 128), lambda i: (0, 0)),
        out_shape=jax.ShapeDtypeStruct((8, 128), jnp.float32),
        scratch_shapes=[pltpu.VMEM((_K, _K), jnp.int16) for _ in range(8)],
    )(qc, dcd, dcn, rv)
    return loss[0, 0]


# speedup vs baseline: 1.3034x; 1.3034x over previous
import functools

import jax
import jax.numpy as jnp
from jax.experimental import pallas as pl
from jax.experimental.pallas import tpu as pltpu

_B, _NQ, _NS, _D = 16, 512, 2048, 64
_K = 128


def _maxes_kernel(q_ref, d_ref, n_ref, rmd_ref, rmn_ref, cmd_ref, cmn_ref):
    qb = q_ref[0]

    def maxes(db, rm_ref, cm_ref):
        s = jax.lax.dot_general(
            db, qb, (((1,), (1,)), ((), ())),
            preferred_element_type=jnp.float32)
        rm_ref[0] = jnp.max(s, axis=0, keepdims=True)
        s2 = jax.lax.dot_general(
            qb, db, (((1,), (1,)), ((), ())),
            preferred_element_type=jnp.float32)
        cm_ref[0] = jnp.max(s2, axis=0, keepdims=True)

    maxes(d_ref[0], rmd_ref, cmd_ref)
    maxes(n_ref[0], rmn_ref, cmn_ref)


def _count16(ref, cand_i32):
    cand = cand_i32.astype(jnp.int16)
    x = (ref[...] >= cand).astype(jnp.int16)
    r = x.shape[0]
    while r > 1:
        h = r // 2
        x = x[:h] + x[h:]
        r = h
    return jnp.sum(x.astype(jnp.int32))


def _bisect16_pair(ref_a, ref_b, need_a, need_b):
    ta = jnp.where(_count16(ref_a, jnp.int32(0)) >= need_a,
                   jnp.int32(0), jnp.int32(-32768))
    tb = jnp.where(_count16(ref_b, jnp.int32(0)) >= need_b,
                   jnp.int32(0), jnp.int32(-32768))

    def body(i, ts):
        ta, tb = ts
        bit = jax.lax.shift_left(jnp.int32(1), jnp.int32(14) - i)
        ca = ta | bit
        cb = tb | bit
        ta = jnp.where(_count16(ref_a, ca) >= need_a, ca, ta)
        tb = jnp.where(_count16(ref_b, cb) >= need_b, cb, tb)
        return ta, tb

    return jax.lax.fori_loop(0, 15, body, (ta, tb))


def _side_keys(qc, dc, hi_ref, lo_ref):
    s = jax.lax.dot_general(
        qc, dc, (((1,), (1,)), ((), ())),
        preferred_element_type=jnp.float32)

    bits = jax.lax.bitcast_convert_type(s, jnp.int32)
    keys = jnp.where(bits < 0, bits ^ jnp.int32(0x7FFFFFFF), bits)
    hi_ref[...] = jax.lax.shift_right_arithmetic(keys, 16).astype(jnp.int16)
    lo_ref[...] = ((keys & jnp.int32(0xFFFF)) ^ jnp.int32(0x8000)).astype(
        jnp.int16)


def _qual_score(rv, tkey):
    rbits = jax.lax.bitcast_convert_type(rv, jnp.int32)
    rkeys = jnp.where(rbits < 0, rbits ^ jnp.int32(0x7FFFFFFF), rbits)
    qual = (rkeys >= tkey).astype(jnp.float32)
    z = jnp.maximum(jnp.sum(qual), 0.001)
    numer = jnp.sum(qual * jnp.maximum(rv, 0.0))
    return numer / z


def _score_kernel(qcd_ref, qcn_ref, dcd_ref, dcn_ref, rvd_ref, rvn_ref,
                  out_ref, hip_ref, lop_ref, hin_ref, lon_ref):
    _side_keys(qcd_ref[0], dcd_ref[0], hip_ref, lop_ref)
    _side_keys(qcn_ref[0], dcn_ref[0], hin_ref, lon_ref)

    kk = jnp.int32(_K)
    hp, hn = _bisect16_pair(hip_ref, hin_ref, kk, kk)
    need_p = kk - _count16(hip_ref, hp + 1)
    need_n = kk - _count16(hin_ref, hn + 1)

    lop_ref[...] = jnp.where(hip_ref[...] == hp.astype(jnp.int16),
                             lop_ref[...], jnp.int16(-32768))
    lon_ref[...] = jnp.where(hin_ref[...] == hn.astype(jnp.int16),
                             lon_ref[...], jnp.int16(-32768))
    lp, ln = _bisect16_pair(lop_ref, lon_ref, need_p, need_n)

    tkey_p = jax.lax.shift_left(hp, 16) | ((lp ^ jnp.int32(0x8000))
                                           & jnp.int32(0xFFFF))
    tkey_n = jax.lax.shift_left(hn, 16) | ((ln ^ jnp.int32(0x8000))
                                           & jnp.int32(0xFFFF))

    pos = _qual_score(rvd_ref[0], tkey_p)
    neg = _qual_score(rvn_ref[0], tkey_n)
    diff = neg - pos
    sp = jnp.maximum(diff, 0.0) + jnp.log1p(jnp.exp(-jnp.abs(diff)))

    @pl.when(pl.program_id(0) == 0)
    def _():
        out_ref[...] = jnp.zeros((8, 128), jnp.float32)

    out_ref[...] += sp / _B


@functools.partial(jax.jit)
def kernel(query_embeddings, doc_embeddings, neg_doc_embeddings):
    rmd, rmn, cmd, cmn = pl.pallas_call(
        _maxes_kernel,
        grid=(_B,),
        in_specs=[
            pl.BlockSpec((1, _NQ, _D), lambda i: (i, 0, 0)),
            pl.BlockSpec((1, _NS, _D), lambda i: (i, 0, 0)),
            pl.BlockSpec((1, _NS, _D), lambda i: (i, 0, 0)),
        ],
        out_specs=[
            pl.BlockSpec((1, 1, _NQ), lambda i: (i, 0, 0)),
            pl.BlockSpec((1, 1, _NQ), lambda i: (i, 0, 0)),
            pl.BlockSpec((1, 1, _NS), lambda i: (i, 0, 0)),
            pl.BlockSpec((1, 1, _NS), lambda i: (i, 0, 0)),
        ],
        out_shape=[
            jax.ShapeDtypeStruct((_B, 1, _NQ), jnp.float32),
            jax.ShapeDtypeStruct((_B, 1, _NQ), jnp.float32),
            jax.ShapeDtypeStruct((_B, 1, _NS), jnp.float32),
            jax.ShapeDtypeStruct((_B, 1, _NS), jnp.float32),
        ],
    )(query_embeddings, doc_embeddings, neg_doc_embeddings)

    rvd, idxd = jax.lax.top_k(rmd[:, 0, :], _K)
    rvn, idxn = jax.lax.top_k(rmn[:, 0, :], _K)
    _, cidxd = jax.lax.top_k(cmd[:, 0, :], _K)
    _, cidxn = jax.lax.top_k(cmn[:, 0, :], _K)
    qcd = jnp.take_along_axis(query_embeddings, idxd[:, :, None], axis=1)
    qcn = jnp.take_along_axis(query_embeddings, idxn[:, :, None], axis=1)
    dcd = jnp.take_along_axis(doc_embeddings, cidxd[:, :, None], axis=1)
    dcn = jnp.take_along_axis(neg_doc_embeddings, cidxn[:, :, None], axis=1)

    loss = pl.pallas_call(
        _score_kernel,
        grid=(_B,),
        in_specs=[
            pl.BlockSpec((1, _K, _D), lambda i: (i, 0, 0)),
            pl.BlockSpec((1, _K, _D), lambda i: (i, 0, 0)),
            pl.BlockSpec((1, _K, _D), lambda i: (i, 0, 0)),
            pl.BlockSpec((1, _K, _D), lambda i: (i, 0, 0)),
            pl.BlockSpec((1, 1, _K), lambda i: (i, 0, 0)),
            pl.BlockSpec((1, 1, _K), lambda i: (i, 0, 0)),
        ],
        out_specs=pl.BlockSpec((8, 128), lambda i: (0, 0)),
        out_shape=jax.ShapeDtypeStruct((8, 128), jnp.float32),
        scratch_shapes=[pltpu.VMEM((_K, _K), jnp.int16),
                        pltpu.VMEM((_K, _K), jnp.int16),
                        pltpu.VMEM((_K, _K), jnp.int16),
                        pltpu.VMEM((_K, _K), jnp.int16)],
    )(qcd, qcn, dcd, dcn, rvd[:, None, :], rvn[:, None, :])
    return loss[0, 0]
